# baseline (device time: 27481 ns/iter reference)
import jax
import jax.numpy as jnp
from jax import lax
from jax.experimental import pallas as pl
from jax.experimental.pallas import tpu as pltpu

N_DEV = 32
N_Z = 4
N_Y = 4
N_C = 8

DZ = (0, 3, 1, 2)
ZSEM = (None, 0, 2, 1)


def kernel(x, w_mat):
    m_per, k = x.shape
    _, n_per = w_mat.shape
    m_glob = N_DEV * m_per

    def body(x_ref, w_ref, out_ref, gx_ref,
             zsend, zrecv, ysend, yrecv, xsend, xrecv):
        my = lax.axis_index("i")
        z = my // N_C
        q = lax.rem(my, N_C)
        yy = q // 2
        p = lax.rem(q, 2)
        xx = jnp.where(lax.rem(yy, 2) == 0, p, 1 - p)

        def q_of(x2, y2):
            return 2 * y2 + jnp.where(lax.rem(y2, 2) == 0, x2, 1 - x2)

        def col_origin(x2, y2, j):
            return lax.rem(z + DZ[j], N_Z) * N_C + q_of(x2, y2)

        zpeers = [lax.rem(z + dz, N_Z) * N_C + q for dz in (1, 2, 3)]
        ypeers = [z * N_C + q_of(xx, lax.rem(yy + dy, N_Y)) for dy in (1, 2, 3)]
        xpartner = z * N_C + q_of(1 - xx, yy)

        def rows_of(origin):
            return pl.ds(origin * m_per, m_per)

        barrier = pltpu.get_barrier_semaphore()
        for nbr in (*zpeers, *ypeers, xpartner):
            pl.semaphore_signal(
                barrier, inc=1,
                device_id=(nbr,), device_id_type=pl.DeviceIdType.MESH,
            )
        pl.semaphore_wait(barrier, 7)

        own_rows = rows_of(my)
        gx_ref[own_rows, :] = x_ref[...]

        started = []

        def send(origin, target, send_bank, recv_bank, idx):
            rdma = pltpu.make_async_remote_copy(
                src_ref=gx_ref.at[rows_of(origin), :],
                dst_ref=gx_ref.at[rows_of(origin), :],
                send_sem=send_bank.at[idx],
                recv_sem=recv_bank.at[idx],
                device_id=(target,),
                device_id_type=pl.DeviceIdType.MESH,
            )
            rdma.start()
            started.append(rdma)

        def wait_recv(recv_bank, idx):
            rdma = pltpu.make_async_remote_copy(
                src_ref=gx_ref.at[own_rows, :],
                dst_ref=gx_ref.at[own_rows, :],
                send_sem=recv_bank.at[idx],
                recv_sem=recv_bank.at[idx],
                device_id=(my,),
                device_id_type=pl.DeviceIdType.MESH,
            )
            rdma.wait_recv()

        for jz, dz in enumerate((1, 2, 3)):
            send(my, zpeers[jz], zsend, zrecv, jz)
        send(my, xpartner, xsend, xrecv, 0)
        for dyi in range(3):
            send(my, ypeers[dyi], ysend, yrecv, dyi * 4)

        for j in (1, 2, 3):
            wait_recv(zrecv, ZSEM[j])
            o = col_origin(xx, yy, j)
            send(o, xpartner, xsend, xrecv, j)
            for dyi in range(3):
                send(o, ypeers[dyi], ysend, yrecv, dyi * 4 + j)

        for i in range(12):
            dy, j = i // 4 + 1, i % 4
            wait_recv(yrecv, i)
            send(col_origin(xx, lax.rem(yy + N_Y - dy, N_Y), j),
                 xpartner, xsend, xrecv, 4 + i)

        for i in range(16):
            wait_recv(xrecv, i)

        out_ref[...] = jnp.dot(
            gx_ref[...], w_ref[...], preferred_element_type=jnp.float32
        )

        for rdma in started:
            rdma.wait_send()

    xb = x.astype(jnp.bfloat16)
    wb = w_mat.astype(jnp.bfloat16)
    return pl.pallas_call(
        body,
        out_shape=jax.ShapeDtypeStruct((m_glob, n_per), jnp.float32),
        in_specs=[
            pl.BlockSpec(memory_space=pltpu.VMEM),
            pl.BlockSpec(memory_space=pltpu.VMEM),
        ],
        out_specs=pl.BlockSpec(memory_space=pltpu.VMEM),
        scratch_shapes=[
            pltpu.VMEM((m_glob, k), jnp.bfloat16),
            pltpu.SemaphoreType.DMA((3,)),
            pltpu.SemaphoreType.DMA((3,)),
            pltpu.SemaphoreType.DMA((12,)),
            pltpu.SemaphoreType.DMA((12,)),
            pltpu.SemaphoreType.DMA((16,)),
            pltpu.SemaphoreType.DMA((16,)),
        ],
        compiler_params=pltpu.CompilerParams(collective_id=0),
    )(xb, wb)


# device time: 22780 ns/iter; 1.2064x vs baseline; 1.2064x over previous
import jax
import jax.numpy as jnp
from jax import lax
from jax.experimental import pallas as pl
from jax.experimental.pallas import tpu as pltpu

N_DEV = 32
N_Z = 4
N_C = 8
N_R = 14
N_L = 14

C2Q = (0, 1, 2, 5, 6, 7, 4, 3)
Q2C = (0, 1, 2, 7, 6, 3, 4, 5)

DZ = (0, 3, 1, 2)
ZSEM = (None, 0, 2, 1)


def _lookup(table, idx):
    r = jnp.int32(table[0])
    for i in range(1, len(table)):
        r = jnp.where(idx == i, jnp.int32(table[i]), r)
    return r


def kernel(x, w_mat):
    m_per, k = x.shape
    _, n_per = w_mat.shape
    m_glob = N_DEV * m_per

    def body(x_ref, w_ref, out_ref, gx_ref,
             zsend, zrecv, rsend, rrecv, lsend, lrecv):
        my = lax.axis_index("i")
        z = my // N_C
        q = lax.rem(my, N_C)
        c = _lookup(Q2C, q)

        def dev_at(c_pos, z_pos):
            return z_pos * N_C + _lookup(C2Q, lax.rem(c_pos, N_C))

        right = dev_at(c + 1, z)
        left = dev_at(c + N_C - 1, z)
        zpeers = [lax.rem(z + dz, N_Z) * N_C + q for dz in (1, 2, 3)]

        def rows_of(origin):
            return pl.ds(origin * m_per, m_per)

        def chunk_origin(dc, j):
            zo = lax.rem(z + DZ[j], N_Z)
            return zo * N_C + _lookup(C2Q, lax.rem(c + N_C + dc, N_C))

        barrier = pltpu.get_barrier_semaphore()
        for nbr in (left, right, *zpeers):
            pl.semaphore_signal(
                barrier, inc=1,
                device_id=(nbr,), device_id_type=pl.DeviceIdType.MESH,
            )
        pl.semaphore_wait(barrier, 5)

        own_rows = rows_of(my)
        gx_ref[own_rows, :] = x_ref[...]

        started = []

        def send(origin, target, send_bank, recv_bank, idx):
            rdma = pltpu.make_async_remote_copy(
                src_ref=gx_ref.at[rows_of(origin), :],
                dst_ref=gx_ref.at[rows_of(origin), :],
                send_sem=send_bank.at[idx],
                recv_sem=recv_bank.at[idx],
                device_id=(target,),
                device_id_type=pl.DeviceIdType.MESH,
            )
            rdma.start()
            started.append(rdma)

        def wait_recv(recv_bank, idx):
            rdma = pltpu.make_async_remote_copy(
                src_ref=gx_ref.at[own_rows, :],
                dst_ref=gx_ref.at[own_rows, :],
                send_sem=recv_bank.at[idx],
                recv_sem=recv_bank.at[idx],
                device_id=(my,),
                device_id_type=pl.DeviceIdType.MESH,
            )
            rdma.wait_recv()

        for j, dz in enumerate((1, 2, 3)):
            send(my, zpeers[j], zsend, zrecv, j)
        send(my, right, rsend, rrecv, 0)
        send(my, left, lsend, lrecv, 0)

        for j in (1, 2, 3):
            wait_recv(zrecv, ZSEM[j])
            o = chunk_origin(0, j)
            send(o, right, rsend, rrecv, j)
            send(o, left, lsend, lrecv, j)

        for i in range(N_R):
            wait_recv(rrecv, i)
            if i <= 9:
                send(chunk_origin(-(i // 4 + 1), i % 4), right,
                     rsend, rrecv, i + 4)
            wait_recv(lrecv, i)
            if i <= 7:
                send(chunk_origin(i // 4 + 1, i % 4), left,
                     lsend, lrecv, i + 4)
            elif i in (10, 11):
                send(chunk_origin(3, i % 4), left, lsend, lrecv, i + 2)

        out_ref[...] = jnp.dot(
            gx_ref[...], w_ref[...], preferred_element_type=jnp.float32
        )

        for rdma in started:
            rdma.wait_send()

    xb = x.astype(jnp.bfloat16)
    wb = w_mat.astype(jnp.bfloat16)
    return pl.pallas_call(
        body,
        out_shape=jax.ShapeDtypeStruct((m_glob, n_per), jnp.float32),
        in_specs=[
            pl.BlockSpec(memory_space=pltpu.VMEM),
            pl.BlockSpec(memory_space=pltpu.VMEM),
        ],
        out_specs=pl.BlockSpec(memory_space=pltpu.VMEM),
        scratch_shapes=[
            pltpu.VMEM((m_glob, k), jnp.bfloat16),
            pltpu.SemaphoreType.DMA((3,)),
            pltpu.SemaphoreType.DMA((3,)),
            pltpu.SemaphoreType.DMA((N_R,)),
            pltpu.SemaphoreType.DMA((N_R,)),
            pltpu.SemaphoreType.DMA((N_L,)),
            pltpu.SemaphoreType.DMA((N_L,)),
        ],
        compiler_params=pltpu.CompilerParams(collective_id=0),
    )(xb, wb)
